# baseline (speedup 1.0000x reference)
import functools

import jax
import jax.numpy as jnp
from jax import lax
from jax.experimental import pallas as pl
from jax.experimental.pallas import tpu as pltpu
from jax.experimental.pallas import tpu_sc as plsc

VOCAB = 100000
EMBED_DIM = 300
PAD_DIM = 304
BATCH = 1024
SENT_LEN = 50

NUM_CORES = 2
NUM_SUBCORES = 16
NUM_WORKERS = NUM_CORES * NUM_SUBCORES

TOTAL = BATCH * SENT_LEN
PER_WORKER = TOTAL // NUM_WORKERS
CHUNK = 80
NCHUNKS = PER_WORKER // CHUNK

_mesh = plsc.VectorSubcoreMesh(core_axis_name="c", subcore_axis_name="s")


@functools.partial(
    pl.kernel,
    mesh=_mesh,
    out_type=jax.ShapeDtypeStruct((TOTAL, PAD_DIM), jnp.float32),
    scratch_types=[
        pltpu.VMEM((NCHUNKS, CHUNK), jnp.int32),
        pltpu.VMEM((CHUNK, PAD_DIM), jnp.float32),
        pltpu.SemaphoreType.DMA,
    ],
    compiler_params=pltpu.CompilerParams(use_tc_tiling_on_sc=False),
)
def _gather_kernel(idx_hbm, table_hbm, out_hbm, idx_v, rows_v, sem):
    wid = lax.axis_index("s") * NUM_CORES + lax.axis_index("c")
    base = wid * PER_WORKER
    pltpu.sync_copy(idx_hbm.at[wid], idx_v)

    def body(j, carry):
        pltpu.async_copy(table_hbm.at[idx_v.at[j]], rows_v, sem).wait()
        pltpu.sync_copy(rows_v, out_hbm.at[pl.ds(base + j * CHUNK, CHUNK)])
        return carry

    lax.fori_loop(0, NCHUNKS, body, 0, unroll=False)


def kernel(x, word_vectors):
    wvp = jnp.pad(word_vectors, ((0, 0), (0, PAD_DIM - EMBED_DIM)))
    idx = x.reshape(NUM_WORKERS, NCHUNKS, CHUNK).astype(jnp.int32)
    out = _gather_kernel(idx, wvp)
    return out[:, :EMBED_DIM].reshape(BATCH, SENT_LEN, EMBED_DIM)


# speedup vs baseline: 2.9460x; 2.9460x over previous
import functools

import jax
import jax.numpy as jnp
from jax import lax
from jax.experimental import pallas as pl
from jax.experimental.pallas import tpu as pltpu
from jax.experimental.pallas import tpu_sc as plsc

VOCAB = 100000
EMBED_DIM = 300
PAD_DIM = 384
BATCH = 1024
SENT_LEN = 50
SENT_PAD = 56

NUM_CORES = 2
NUM_SUBCORES = 16
NUM_WORKERS = NUM_CORES * NUM_SUBCORES
ROWS_PER_WORKER = BATCH // NUM_WORKERS

VB = 512
NVB = (VOCAB + VB - 1) // VB
IB = 128
JB = 8

_mesh = plsc.VectorSubcoreMesh(core_axis_name="c", subcore_axis_name="s")


def _t_in_kernel(wvt_ref, out_ref):
    v = wvt_ref[...]
    z = jnp.zeros((PAD_DIM - EMBED_DIM, VB), jnp.float32)
    out_ref[...] = jnp.concatenate([v, z], axis=0).T


_transpose_in = pl.pallas_call(
    _t_in_kernel,
    grid=(NVB,),
    in_specs=[pl.BlockSpec((EMBED_DIM, VB), lambda i: (0, i))],
    out_specs=pl.BlockSpec((VB, PAD_DIM), lambda i: (i, 0)),
    out_shape=jax.ShapeDtypeStruct((NVB * VB, PAD_DIM), jnp.float32),
)


@functools.partial(
    pl.kernel,
    mesh=_mesh,
    out_type=jax.ShapeDtypeStruct((BATCH, SENT_PAD, PAD_DIM), jnp.float32),
    scratch_types=[
        pltpu.VMEM((ROWS_PER_WORKER, SENT_PAD), jnp.int32),
        pltpu.VMEM((SENT_PAD, PAD_DIM), jnp.float32),
        pltpu.SemaphoreType.DMA,
    ],
)
def _gather_kernel(idx_hbm, table_hbm, out_hbm, idx_v, rows_v, sem):
    wid = lax.axis_index("s") * NUM_CORES + lax.axis_index("c")
    pltpu.sync_copy(idx_hbm.at[wid], idx_v)

    def body(b, carry):
        pltpu.async_copy(table_hbm.at[idx_v.at[b]], rows_v, sem).wait()
        pltpu.sync_copy(rows_v, out_hbm.at[wid * ROWS_PER_WORKER + b])
        return carry

    lax.fori_loop(0, ROWS_PER_WORKER, body, 0, unroll=False)


def _t_out_kernel(rows_ref, out_ref):
    for j in range(JB):
        out_ref[j] = rows_ref[:, j, :].T[:EMBED_DIM]


_transpose_out = pl.pallas_call(
    _t_out_kernel,
    grid=(BATCH // IB, (SENT_LEN + JB - 1) // JB),
    in_specs=[pl.BlockSpec((IB, JB, PAD_DIM), lambda i, j: (i, j, 0))],
    out_specs=pl.BlockSpec((JB, EMBED_DIM, IB), lambda i, j: (j, 0, i)),
    out_shape=jax.ShapeDtypeStruct((SENT_LEN, EMBED_DIM, BATCH), jnp.float32),
)


def kernel(x, word_vectors):
    wvt = jnp.swapaxes(word_vectors, 0, 1)
    table = _transpose_in(wvt)
    idx3 = x.reshape(NUM_WORKERS, ROWS_PER_WORKER, SENT_LEN).astype(jnp.int32)
    fill = (
        jnp.arange(SENT_PAD - SENT_LEN, dtype=jnp.int32)[None, None, :]
        + 8 * jnp.arange(ROWS_PER_WORKER, dtype=jnp.int32)[None, :, None]
        + 256 * jnp.arange(NUM_WORKERS, dtype=jnp.int32)[:, None, None]
    )
    idxp = jnp.concatenate([idx3, fill], axis=2)
    rows = _gather_kernel(idxp, table)
    g = _transpose_out(rows)
    return jnp.transpose(g, (2, 0, 1))


# speedup vs baseline: 3.7250x; 1.2644x over previous
import functools

import jax
import jax.numpy as jnp
from jax import lax
from jax.experimental import pallas as pl
from jax.experimental.pallas import tpu as pltpu
from jax.experimental.pallas import tpu_sc as plsc

VOCAB = 100000
EMBED_DIM = 300
PAD_DIM = 384
BATCH = 1024
SENT_LEN = 50
SENT_PAD = 56

NUM_CORES = 2
NUM_SUBCORES = 16
NUM_WORKERS = NUM_CORES * NUM_SUBCORES
ROWS_PER_WORKER = BATCH // NUM_WORKERS

VB = 1024
NVB = (VOCAB + VB - 1) // VB
IB = 128

_mesh = plsc.VectorSubcoreMesh(core_axis_name="c", subcore_axis_name="s")


def _t_in_kernel(wvt_ref, out_ref):
    v = wvt_ref[...]
    z = jnp.zeros((PAD_DIM - EMBED_DIM, VB), jnp.float32)
    out_ref[...] = jnp.concatenate([v, z], axis=0).T


_transpose_in = pl.pallas_call(
    _t_in_kernel,
    grid=(NVB,),
    in_specs=[pl.BlockSpec((EMBED_DIM, VB), lambda i: (0, i))],
    out_specs=pl.BlockSpec((VB, PAD_DIM), lambda i: (i, 0)),
    out_shape=jax.ShapeDtypeStruct((NVB * VB, PAD_DIM), jnp.float32),
)


@functools.partial(
    pl.kernel,
    mesh=_mesh,
    out_type=jax.ShapeDtypeStruct((BATCH, SENT_PAD, PAD_DIM), jnp.float32),
    scratch_types=[
        pltpu.VMEM((ROWS_PER_WORKER, SENT_PAD), jnp.int32),
        pltpu.VMEM((SENT_PAD, PAD_DIM), jnp.float32),
        pltpu.SemaphoreType.DMA,
    ],
)
def _gather_kernel(idx_hbm, table_hbm, out_hbm, idx_v, rows_v, sem):
    wid = lax.axis_index("s") * NUM_CORES + lax.axis_index("c")
    pltpu.sync_copy(idx_hbm.at[wid], idx_v)

    def body(b, carry):
        pltpu.async_copy(table_hbm.at[idx_v.at[b]], rows_v, sem).wait()
        pltpu.sync_copy(rows_v, out_hbm.at[wid * ROWS_PER_WORKER + b])
        return carry

    lax.fori_loop(0, ROWS_PER_WORKER, body, 0, unroll=False)


def _t_out_kernel(rows_ref, out_ref):
    for j in range(SENT_LEN):
        out_ref[j] = rows_ref[:, j, :].T[:EMBED_DIM]


_transpose_out = pl.pallas_call(
    _t_out_kernel,
    grid=(BATCH // IB,),
    in_specs=[pl.BlockSpec((IB, SENT_PAD, PAD_DIM), lambda i: (i, 0, 0))],
    out_specs=pl.BlockSpec((SENT_LEN, EMBED_DIM, IB), lambda i: (0, 0, i)),
    out_shape=jax.ShapeDtypeStruct((SENT_LEN, EMBED_DIM, BATCH), jnp.float32),
)


def kernel(x, word_vectors):
    wvt = jnp.swapaxes(word_vectors, 0, 1)
    table = _transpose_in(wvt)
    idx3 = x.reshape(NUM_WORKERS, ROWS_PER_WORKER, SENT_LEN).astype(jnp.int32)
    fill = (
        jnp.arange(SENT_PAD - SENT_LEN, dtype=jnp.int32)[None, None, :]
        + 8 * jnp.arange(ROWS_PER_WORKER, dtype=jnp.int32)[None, :, None]
        + 256 * jnp.arange(NUM_WORKERS, dtype=jnp.int32)[:, None, None]
    )
    idxp = jnp.concatenate([idx3, fill], axis=2)
    rows = _gather_kernel(idxp, table)
    g = _transpose_out(rows)
    return jnp.transpose(g, (2, 0, 1))


# speedup vs baseline: 4.3986x; 1.1808x over previous
import functools

import jax
import jax.numpy as jnp
from jax import lax
from jax.experimental import pallas as pl
from jax.experimental.pallas import tpu as pltpu
from jax.experimental.pallas import tpu_sc as plsc

VOCAB = 100000
EMBED_DIM = 300
PAD_DIM = 384
BATCH = 1024
SENT_LEN = 50
SENT_PAD = 56

NUM_CORES = 2
NUM_SUBCORES = 16
NUM_WORKERS = NUM_CORES * NUM_SUBCORES
ROWS_PER_WORKER = BATCH // NUM_WORKERS

VB = 2048
NVB = (VOCAB + VB - 1) // VB
IB = 128

_mesh = plsc.VectorSubcoreMesh(core_axis_name="c", subcore_axis_name="s")


def _t_in_kernel(wvt_ref, out_ref):
    v = wvt_ref[...]
    z = jnp.zeros((PAD_DIM - EMBED_DIM, VB), jnp.float32)
    out_ref[...] = jnp.concatenate([v, z], axis=0).T


_transpose_in = pl.pallas_call(
    _t_in_kernel,
    grid=(NVB,),
    in_specs=[pl.BlockSpec((EMBED_DIM, VB), lambda i: (0, i))],
    out_specs=pl.BlockSpec((VB, PAD_DIM), lambda i: (i, 0)),
    out_shape=jax.ShapeDtypeStruct((NVB * VB, PAD_DIM), jnp.float32),
)


@functools.partial(
    pl.kernel,
    mesh=_mesh,
    out_type=jax.ShapeDtypeStruct((BATCH, SENT_PAD, PAD_DIM), jnp.float32),
    scratch_types=[
        pltpu.VMEM((ROWS_PER_WORKER, SENT_PAD), jnp.int32),
        pltpu.VMEM((2, SENT_PAD, PAD_DIM), jnp.float32),
        pltpu.SemaphoreType.DMA,
        pltpu.SemaphoreType.DMA,
    ],
)
def _gather_kernel(idx_hbm, table_hbm, out_hbm, idx_v, rows_v, sem_g, sem_w):
    wid = lax.axis_index("s") * NUM_CORES + lax.axis_index("c")
    base = wid * ROWS_PER_WORKER
    pltpu.sync_copy(idx_hbm.at[wid], idx_v)

    pltpu.async_copy(table_hbm.at[idx_v.at[0]], rows_v.at[0], sem_g)

    def body(b, carry):
        pltpu.make_async_copy(out_hbm.at[0], rows_v.at[0], sem_g).wait()

        @pl.when(b >= 1)
        def _():
            pltpu.make_async_copy(out_hbm.at[0], rows_v.at[0], sem_w).wait()

        @pl.when(b + 1 < ROWS_PER_WORKER)
        def _():
            nxt = lax.rem(b + 1, 2)
            pltpu.async_copy(table_hbm.at[idx_v.at[b + 1]], rows_v.at[nxt],
                             sem_g)

        cur = lax.rem(b, 2)
        pltpu.async_copy(rows_v.at[cur], out_hbm.at[base + b], sem_w)
        return carry

    lax.fori_loop(0, ROWS_PER_WORKER, body, 0, unroll=False)
    pltpu.make_async_copy(out_hbm.at[0], rows_v.at[0], sem_w).wait()


def _t_out_kernel(rows_ref, out_ref):
    for j in range(SENT_LEN):
        out_ref[j] = rows_ref[:, j, :].T[:EMBED_DIM]


_transpose_out = pl.pallas_call(
    _t_out_kernel,
    grid=(BATCH // IB,),
    in_specs=[pl.BlockSpec((IB, SENT_PAD, PAD_DIM), lambda i: (i, 0, 0))],
    out_specs=pl.BlockSpec((SENT_LEN, EMBED_DIM, IB), lambda i: (0, 0, i)),
    out_shape=jax.ShapeDtypeStruct((SENT_LEN, EMBED_DIM, BATCH), jnp.float32),
)


def kernel(x, word_vectors):
    wvt = jnp.swapaxes(word_vectors, 0, 1)
    table = _transpose_in(wvt)
    idx3 = x.reshape(NUM_WORKERS, ROWS_PER_WORKER, SENT_LEN).astype(jnp.int32)
    fill = (
        jnp.arange(SENT_PAD - SENT_LEN, dtype=jnp.int32)[None, None, :]
        + 8 * jnp.arange(ROWS_PER_WORKER, dtype=jnp.int32)[None, :, None]
        + 256 * jnp.arange(NUM_WORKERS, dtype=jnp.int32)[:, None, None]
    )
    idxp = jnp.concatenate([idx3, fill], axis=2)
    rows = _gather_kernel(idxp, table)
    g = _transpose_out(rows)
    return jnp.transpose(g, (2, 0, 1))


# speedup vs baseline: 4.5972x; 1.0451x over previous
import functools

import jax
import jax.numpy as jnp
from jax import lax
from jax.experimental import pallas as pl
from jax.experimental.pallas import tpu as pltpu
from jax.experimental.pallas import tpu_sc as plsc

VOCAB = 100000
EMBED_DIM = 300
PAD_DIM = 384
BATCH = 1024
SENT_LEN = 50
SENT_PAD = 56

NUM_CORES = 2
NUM_SUBCORES = 16
NUM_WORKERS = NUM_CORES * NUM_SUBCORES
ROWS_PER_WORKER = BATCH // NUM_WORKERS

VB = 2048
NVB = (VOCAB + VB - 1) // VB
IB = 128

_mesh = plsc.VectorSubcoreMesh(core_axis_name="c", subcore_axis_name="s")


def _t_in_kernel(wvt_ref, out_ref):
    v = wvt_ref[...]
    z = jnp.zeros((PAD_DIM - EMBED_DIM, VB), jnp.float32)
    out_ref[...] = jnp.concatenate([v, z], axis=0).T


_transpose_in = pl.pallas_call(
    _t_in_kernel,
    grid=(NVB,),
    in_specs=[pl.BlockSpec((EMBED_DIM, VB), lambda i: (0, i))],
    out_specs=pl.BlockSpec((VB, PAD_DIM), lambda i: (i, 0)),
    out_shape=jax.ShapeDtypeStruct((NVB * VB, PAD_DIM), jnp.float32),
)


@functools.partial(
    pl.kernel,
    mesh=_mesh,
    out_type=jax.ShapeDtypeStruct((BATCH, SENT_PAD, PAD_DIM), jnp.float32),
    scratch_types=[
        pltpu.VMEM((ROWS_PER_WORKER, SENT_PAD), jnp.int32),
        pltpu.VMEM((4, SENT_PAD, PAD_DIM), jnp.float32),
        pltpu.SemaphoreType.DMA,
        pltpu.SemaphoreType.DMA,
    ],
)
def _gather_kernel(idx_hbm, table_hbm, out_hbm, idx_v, rows_v, sem_g, sem_w):
    wid = lax.axis_index("s") * NUM_CORES + lax.axis_index("c")
    base = wid * ROWS_PER_WORKER
    pltpu.sync_copy(idx_hbm.at[wid], idx_v)

    pltpu.async_copy(table_hbm.at[idx_v.at[0]], rows_v.at[0], sem_g)
    pltpu.async_copy(table_hbm.at[idx_v.at[1]], rows_v.at[1], sem_g)

    def body(b, carry):
        pltpu.make_async_copy(out_hbm.at[0], rows_v.at[0], sem_g).wait()

        @pl.when(b >= 2)
        def _():
            pltpu.make_async_copy(out_hbm.at[0], rows_v.at[0], sem_w).wait()

        @pl.when(b + 2 < ROWS_PER_WORKER)
        def _():
            nxt = lax.rem(b + 2, 4)
            pltpu.async_copy(table_hbm.at[idx_v.at[b + 2]], rows_v.at[nxt],
                             sem_g)

        cur = lax.rem(b, 4)
        pltpu.async_copy(rows_v.at[cur], out_hbm.at[base + b], sem_w)
        return carry

    lax.fori_loop(0, ROWS_PER_WORKER, body, 0, unroll=False)
    pltpu.make_async_copy(out_hbm.at[0], rows_v.at[0], sem_w).wait()
    pltpu.make_async_copy(out_hbm.at[0], rows_v.at[0], sem_w).wait()


def _t_out_kernel(rows_ref, out_ref):
    for j in range(SENT_LEN):
        out_ref[j] = rows_ref[:, j, :].T[:EMBED_DIM]


_transpose_out = pl.pallas_call(
    _t_out_kernel,
    grid=(BATCH // IB,),
    in_specs=[pl.BlockSpec((IB, SENT_PAD, PAD_DIM), lambda i: (i, 0, 0))],
    out_specs=pl.BlockSpec((SENT_LEN, EMBED_DIM, IB), lambda i: (0, 0, i)),
    out_shape=jax.ShapeDtypeStruct((SENT_LEN, EMBED_DIM, BATCH), jnp.float32),
)


def kernel(x, word_vectors):
    wvt = jnp.swapaxes(word_vectors, 0, 1)
    table = _transpose_in(wvt)
    idx3 = x.reshape(NUM_WORKERS, ROWS_PER_WORKER, SENT_LEN).astype(jnp.int32)
    fill = (
        jnp.arange(SENT_PAD - SENT_LEN, dtype=jnp.int32)[None, None, :]
        + 8 * jnp.arange(ROWS_PER_WORKER, dtype=jnp.int32)[None, :, None]
        + 256 * jnp.arange(NUM_WORKERS, dtype=jnp.int32)[:, None, None]
    )
    idxp = jnp.concatenate([idx3, fill], axis=2)
    rows = _gather_kernel(idxp, table)
    g = _transpose_out(rows)
    return jnp.transpose(g, (2, 0, 1))


# speedup vs baseline: 4.6281x; 1.0067x over previous
import functools

import jax
import jax.numpy as jnp
from jax import lax
from jax.experimental import pallas as pl
from jax.experimental.pallas import tpu as pltpu
from jax.experimental.pallas import tpu_sc as plsc

VOCAB = 100000
EMBED_DIM = 300
PAD_DIM = 384
BATCH = 1024
SENT_LEN = 50
SENT_PAD = 56

NUM_CORES = 2
NUM_SUBCORES = 16
NUM_WORKERS = NUM_CORES * NUM_SUBCORES
HALF = BATCH // 2
ROWS_PER_WORKER = HALF // NUM_WORKERS

VB = 2048
NVB = (VOCAB + VB - 1) // VB
IB = 128

_mesh = plsc.VectorSubcoreMesh(core_axis_name="c", subcore_axis_name="s")


def _t_in_kernel(wvt_ref, out_ref):
    v = wvt_ref[...]
    z = jnp.zeros((PAD_DIM - EMBED_DIM, VB), jnp.float32)
    out_ref[...] = jnp.concatenate([v, z], axis=0).T


_transpose_in = pl.pallas_call(
    _t_in_kernel,
    grid=(NVB,),
    in_specs=[pl.BlockSpec((EMBED_DIM, VB), lambda i: (0, i))],
    out_specs=pl.BlockSpec((VB, PAD_DIM), lambda i: (i, 0)),
    out_shape=jax.ShapeDtypeStruct((NVB * VB, PAD_DIM), jnp.float32),
)


@functools.partial(
    pl.kernel,
    mesh=_mesh,
    out_type=jax.ShapeDtypeStruct((HALF, SENT_PAD, PAD_DIM), jnp.float32),
    scratch_types=[
        pltpu.VMEM((ROWS_PER_WORKER, SENT_PAD), jnp.int32),
        pltpu.VMEM((4, SENT_PAD, PAD_DIM), jnp.float32),
        pltpu.SemaphoreType.DMA,
        pltpu.SemaphoreType.DMA,
    ],
)
def _gather_kernel(idx_hbm, table_hbm, out_hbm, idx_v, rows_v, sem_g, sem_w):
    wid = lax.axis_index("s") * NUM_CORES + lax.axis_index("c")
    base = wid * ROWS_PER_WORKER
    pltpu.sync_copy(idx_hbm.at[wid], idx_v)

    pltpu.async_copy(table_hbm.at[idx_v.at[0]], rows_v.at[0], sem_g)
    pltpu.async_copy(table_hbm.at[idx_v.at[1]], rows_v.at[1], sem_g)

    def body(b, carry):
        pltpu.make_async_copy(out_hbm.at[0], rows_v.at[0], sem_g).wait()

        @pl.when(b >= 2)
        def _():
            pltpu.make_async_copy(out_hbm.at[0], rows_v.at[0], sem_w).wait()

        @pl.when(b + 2 < ROWS_PER_WORKER)
        def _():
            nxt = lax.rem(b + 2, 4)
            pltpu.async_copy(table_hbm.at[idx_v.at[b + 2]], rows_v.at[nxt],
                             sem_g)

        cur = lax.rem(b, 4)
        pltpu.async_copy(rows_v.at[cur], out_hbm.at[base + b], sem_w)
        return carry

    lax.fori_loop(0, ROWS_PER_WORKER, body, 0, unroll=False)
    pltpu.make_async_copy(out_hbm.at[0], rows_v.at[0], sem_w).wait()
    pltpu.make_async_copy(out_hbm.at[0], rows_v.at[0], sem_w).wait()


def _t_out_a(rows_ref, out_ref):
    for j in range(SENT_LEN):
        out_ref[j] = rows_ref[:, j, :].T[:EMBED_DIM]


def _t_out_b(rows_ref, base_ref, out_ref):
    del base_ref
    for j in range(SENT_LEN):
        out_ref[j] = rows_ref[:, j, :].T[:EMBED_DIM]


_transpose_out_a = pl.pallas_call(
    _t_out_a,
    grid=(HALF // IB,),
    in_specs=[pl.BlockSpec((IB, SENT_PAD, PAD_DIM), lambda i: (i, 0, 0))],
    out_specs=pl.BlockSpec((SENT_LEN, EMBED_DIM, IB), lambda i: (0, 0, i)),
    out_shape=jax.ShapeDtypeStruct((SENT_LEN, EMBED_DIM, BATCH), jnp.float32),
)

_transpose_out_b = pl.pallas_call(
    _t_out_b,
    grid=(HALF // IB,),
    in_specs=[
        pl.BlockSpec((IB, SENT_PAD, PAD_DIM), lambda i: (i, 0, 0)),
        pl.BlockSpec(memory_space=pl.ANY),
    ],
    out_specs=pl.BlockSpec((SENT_LEN, EMBED_DIM, IB),
                           lambda i: (0, 0, i + HALF // IB)),
    out_shape=jax.ShapeDtypeStruct((SENT_LEN, EMBED_DIM, BATCH), jnp.float32),
    input_output_aliases={1: 0},
)


def kernel(x, word_vectors):
    wvt = jnp.swapaxes(word_vectors, 0, 1)
    table = _transpose_in(wvt)
    idx3 = x.reshape(2, NUM_WORKERS, ROWS_PER_WORKER, SENT_LEN).astype(
        jnp.int32)
    fill = (
        jnp.arange(SENT_PAD - SENT_LEN, dtype=jnp.int32)[None, None, None, :]
        + 8 * jnp.arange(ROWS_PER_WORKER, dtype=jnp.int32)[None, None, :, None]
        + 128 * jnp.arange(NUM_WORKERS, dtype=jnp.int32)[None, :, None, None]
        + 4096 * jnp.arange(2, dtype=jnp.int32)[:, None, None, None]
    )
    idxp = jnp.concatenate([idx3, fill], axis=3)
    rows_a = _gather_kernel(idxp[0], table)
    rows_b = _gather_kernel(idxp[1], table)
    g = _transpose_out_a(rows_a)
    g = _transpose_out_b(rows_b, g)
    return jnp.transpose(g, (2, 0, 1))
